# baseline (speedup 1.0000x reference)
import functools

import jax
import jax.numpy as jnp
from jax import lax
from jax.experimental import pallas as pl
from jax.experimental.pallas import tpu as pltpu
from jax.experimental.pallas import tpu_sc as plsc

VOCAB_N = 100000
CTX_N = 20
EMB_N = 64
HID_N = 512
BATCH_N = 1024

_NC = 2
_NS = 16
_NW = _NC * _NS
_TOT = BATCH_N * CTX_N
_CHUNK = 128
_NROWS = _TOT // _CHUNK
_CPW = _NROWS // _NW


@functools.cache
def _sc_gather_fn():
    mesh = plsc.VectorSubcoreMesh(core_axis_name="c", subcore_axis_name="s")

    @functools.partial(
        pl.kernel,
        mesh=mesh,
        out_type=jax.ShapeDtypeStruct((_NW, _CPW, _CHUNK, EMB_N), jnp.float32),
        scratch_types=[
            pltpu.VMEM((_CPW, _CHUNK), jnp.int32),
            pltpu.VMEM((_CPW, _CHUNK, EMB_N), jnp.float32),
            pltpu.SemaphoreType.DMA,
        ],
        compiler_params=pltpu.CompilerParams(use_tc_tiling_on_sc=False),
    )
    def _sc_gather(table_hbm, idx_hbm, out_hbm, idx_v, rows_v, sem):
        wid = lax.axis_index("s") * _NC + lax.axis_index("c")
        pltpu.sync_copy(idx_hbm.at[wid], idx_v)
        copies = [
            pltpu.async_copy(table_hbm.at[idx_v.at[i]], rows_v.at[i], sem)
            for i in range(_CPW)
        ]
        for c in copies:
            c.wait()
        pltpu.sync_copy(rows_v, out_hbm.at[wid])

    return _sc_gather


_BN = 2048
_NBLK = 48
_TAIL0 = _NBLK * _BN
_NBUF = 3
_RIN = 4
_ROUT = 8


def _proj_body(flat_ref, w1_ref, b1_ref, w2_hbm, b2_hbm,
               out_hbm, h_ref, w2_buf, b2_buf, out_buf, in_sems, out_sems):
    def in_copies(j, s):
        col = j * _BN
        cps = [
            pltpu.make_async_copy(
                w2_hbm.at[pl.ds(r * 128, 128), pl.ds(col, _BN)],
                w2_buf.at[s, pl.ds(r * 128, 128), :],
                in_sems.at[s])
            for r in range(_RIN)
        ]
        cps.append(pltpu.make_async_copy(
            b2_hbm.at[:, pl.ds(col, _BN)], b2_buf.at[s], in_sems.at[s]))
        return cps

    def out_copies(j, s):
        col = j * _BN
        return [
            pltpu.make_async_copy(
                out_buf.at[s, pl.ds(r * 128, 128), :],
                out_hbm.at[pl.ds(r * 128, 128), pl.ds(col, _BN)],
                out_sems.at[s])
            for r in range(_ROUT)
        ]

    def start_next(j, s):
        for c in in_copies(j, s):
            c.start()

    start_next(0, 0)
    start_next(1, 1)

    h = jnp.dot(flat_ref[...].astype(jnp.bfloat16),
                w1_ref[...].astype(jnp.bfloat16),
                preferred_element_type=jnp.float32)
    h_ref[...] = jnp.maximum(h + b1_ref[...], 0.0).astype(jnp.bfloat16)

    def step(j, carry):
        s = lax.rem(j, _NBUF)

        @pl.when(j + 2 < _NBLK)
        def _():
            start_next(j + 2, lax.rem(j + 2, _NBUF))

        for c in in_copies(j, s):
            c.wait()

        @pl.when(j >= _NBUF)
        def _():
            for c in out_copies(j, s):
                c.wait()

        acc = jnp.dot(h_ref[...], w2_buf[s].astype(jnp.bfloat16),
                      preferred_element_type=jnp.float32)
        out_buf[s] = acc + b2_buf[s]
        for c in out_copies(j, s):
            c.start()
        return carry

    lax.fori_loop(0, _NBLK, step, 0)

    for jj in range(_NBLK - _NBUF, _NBLK):
        for c in out_copies(jj, jj % _NBUF):
            c.wait()


_proj = pl.pallas_call(
    _proj_body,
    in_specs=[
        pl.BlockSpec(memory_space=pltpu.VMEM),
        pl.BlockSpec(memory_space=pltpu.VMEM),
        pl.BlockSpec(memory_space=pltpu.VMEM),
        pl.BlockSpec(memory_space=pl.ANY),
        pl.BlockSpec(memory_space=pl.ANY),
    ],
    out_specs=[
        pl.BlockSpec(memory_space=pl.ANY),
        pl.BlockSpec(memory_space=pltpu.VMEM),
    ],
    out_shape=[
        jax.ShapeDtypeStruct((BATCH_N, VOCAB_N), jnp.float32),
        jax.ShapeDtypeStruct((BATCH_N, HID_N), jnp.bfloat16),
    ],
    scratch_shapes=[
        pltpu.VMEM((_NBUF, HID_N, _BN), jnp.float32),
        pltpu.VMEM((_NBUF, 1, _BN), jnp.float32),
        pltpu.VMEM((_NBUF, BATCH_N, _BN), jnp.float32),
        pltpu.SemaphoreType.DMA((_NBUF,)),
        pltpu.SemaphoreType.DMA((_NBUF,)),
    ],
    compiler_params=pltpu.CompilerParams(vmem_limit_bytes=60 * 1024 * 1024),
)


def _tail_body(prev_ref, h_ref, w2_ref, b2_ref, out_ref):
    del prev_ref
    out_ref[...] = (
        jnp.dot(h_ref[...], w2_ref[...].astype(jnp.bfloat16),
                preferred_element_type=jnp.float32)
        + b2_ref[...]
    )


_tail = pl.pallas_call(
    _tail_body,
    grid=(1,),
    in_specs=[
        pl.BlockSpec(memory_space=pl.ANY),
        pl.BlockSpec((BATCH_N, HID_N), lambda i: (0, 0)),
        pl.BlockSpec((HID_N, _BN), lambda i: (0, _NBLK)),
        pl.BlockSpec((1, _BN), lambda i: (0, _NBLK)),
    ],
    out_specs=pl.BlockSpec((BATCH_N, _BN), lambda i: (0, _NBLK)),
    out_shape=jax.ShapeDtypeStruct((BATCH_N, VOCAB_N), jnp.float32),
    input_output_aliases={0: 0},
)


def kernel(x, emb_table, W1, b1, W2, b2):
    idx = x.astype(jnp.int32).reshape(_NW, _CPW, _CHUNK)
    flat4 = _sc_gather_fn()(emb_table, idx)
    flat = flat4.reshape(BATCH_N, CTX_N * EMB_N)
    out_main, h = _proj(flat, W1, b1.reshape(1, HID_N),
                        W2, b2.reshape(1, VOCAB_N))
    return _tail(out_main, h, W2, b2.reshape(1, VOCAB_N))
